# Initial kernel scaffold; baseline (speedup 1.0000x reference)
import jax
import jax.numpy as jnp
from jax.experimental import pallas as pl


def kernel(item_seq, pq_codes, emb_table):
    raise NotImplementedError("write your pallas kernel here")


# speedup vs baseline: 6.6867x; 6.6867x over previous
import functools

import jax
import jax.numpy as jnp
from jax import lax
from jax.experimental import pallas as pl
from jax.experimental.pallas import tpu as pltpu
from jax.experimental.pallas import tpu_sc as plsc

CODE_DIM = 32
OUT_DIM = 64
LANES = 16
DCH = OUT_DIM // LANES
CHUNK = 64
SUB = 8


def _sc_body(num_workers, n_queries, item_hbm, pq_hbm, emb_hbm, out_hbm,
             ids_v, codes_v, rows_v, out_v, sem_pq, sem_emb):
    wid = lax.axis_index("s") * 2 + lax.axis_index("c")
    qpw = n_queries // num_workers
    base = wid * qpw

    def chunk_body(ci, _):
        start = base + ci * CHUNK
        pltpu.sync_copy(item_hbm.at[pl.ds(start, CHUNK)], ids_v)
        pltpu.async_copy(pq_hbm.at[ids_v], codes_v, sem_pq).wait()

        def sub_body(j, _):
            qb = j * SUB
            cps = [
                pltpu.async_copy(emb_hbm.at[codes_v.at[qb + k]],
                                 rows_v.at[k], sem_emb)
                for k in range(SUB)
            ]
            for cp in cps:
                cp.wait()
            for k in range(SUB):
                def acc_body(c, acc):
                    return tuple(
                        acc[d] + rows_v[k, c, pl.ds(d * LANES, LANES)]
                        for d in range(DCH)
                    )
                acc = lax.fori_loop(
                    0, CODE_DIM, acc_body,
                    tuple(jnp.zeros((LANES,), jnp.float32)
                          for _ in range(DCH)))
                for d in range(DCH):
                    out_v[qb + k, pl.ds(d * LANES, LANES)] = (
                        acc[d] * (1.0 / CODE_DIM))
            return 0

        lax.fori_loop(0, CHUNK // SUB, sub_body, 0)
        pltpu.sync_copy(out_v, out_hbm.at[pl.ds(start, CHUNK)])
        return 0

    lax.fori_loop(0, qpw // CHUNK, chunk_body, 0)


def kernel(item_seq, pq_codes, emb_table):
    batch, hist = item_seq.shape
    n_queries = batch * hist
    info = plsc.get_sparse_core_info()
    num_workers = info.num_cores * info.num_subcores
    assert n_queries % (num_workers * CHUNK) == 0

    mesh = plsc.VectorSubcoreMesh(core_axis_name="c", subcore_axis_name="s")
    run = pl.kernel(
        functools.partial(_sc_body, num_workers, n_queries),
        out_type=jax.ShapeDtypeStruct((n_queries, OUT_DIM), jnp.float32),
        mesh=mesh,
        scratch_types=[
            pltpu.VMEM((CHUNK,), jnp.int32),
            pltpu.VMEM((CHUNK, CODE_DIM), jnp.int32),
            pltpu.VMEM((SUB, CODE_DIM, OUT_DIM), jnp.float32),
            pltpu.VMEM((CHUNK, OUT_DIM), jnp.float32),
            pltpu.SemaphoreType.DMA,
            pltpu.SemaphoreType.DMA,
        ],
        compiler_params=pltpu.CompilerParams(use_tc_tiling_on_sc=False),
    )
    out = run(item_seq.reshape(n_queries), pq_codes, emb_table)
    return out.reshape(batch, hist, OUT_DIM)


# speedup vs baseline: 9.2248x; 1.3796x over previous
import functools

import jax
import jax.numpy as jnp
from jax import lax
from jax.experimental import pallas as pl
from jax.experimental.pallas import tpu as pltpu
from jax.experimental.pallas import tpu_sc as plsc

CODE_DIM = 32
OUT_DIM = 64
LANES = 16
DCH = OUT_DIM // LANES
STAGE = 64
R = 16
UNROLL = 4


def _pool_round(codes_v, rows_v, out_v, out_hbm, emb_hbm, sem, r_start):
    for k in range(R):
        def acc_body(cc, acc):
            new = list(acc)
            for u in range(UNROLL):
                c = cc * UNROLL + u
                for d in range(DCH):
                    new[d] = new[d] + rows_v[k, c, pl.ds(d * LANES, LANES)]
            return tuple(new)
        acc = lax.fori_loop(
            0, CODE_DIM // UNROLL, acc_body,
            tuple(jnp.zeros((LANES,), jnp.float32) for _ in range(DCH)))
        for d in range(DCH):
            out_v[k, pl.ds(d * LANES, LANES)] = acc[d] * (1.0 / CODE_DIM)
    pltpu.sync_copy(out_v, out_hbm.at[pl.ds(r_start, R)])


def _fire(emb_hbm, codes_v, rows_v, sem, r):
    for k in range(R):
        pltpu.async_copy(
            emb_hbm.at[codes_v.at[r * R + k]], rows_v.at[k], sem)


def _drain(emb_hbm, codes_v, rows_v, sem):
    for k in range(R):
        pltpu.make_async_copy(
            emb_hbm.at[codes_v.at[k]], rows_v.at[k], sem).wait()


def _sc_body(num_workers, n_queries, item_hbm, pq_hbm, emb_hbm, out_hbm,
             ids_v, codes_v, rows_a, rows_b, out_v,
             sem_stage, sem_a, sem_b):
    wid = lax.axis_index("s") * 2 + lax.axis_index("c")
    qpw = n_queries // num_workers
    base = wid * qpw
    nrounds = qpw // R

    pltpu.sync_copy(item_hbm.at[pl.ds(base, qpw)], ids_v)
    cps = [
        pltpu.async_copy(pq_hbm.at[ids_v.at[pl.ds(i * STAGE, STAGE)]],
                         codes_v.at[pl.ds(i * STAGE, STAGE), :], sem_stage)
        for i in range(qpw // STAGE)
    ]
    for cp in cps:
        cp.wait()

    _fire(emb_hbm, codes_v, rows_a, sem_a, 0)
    _fire(emb_hbm, codes_v, rows_b, sem_b, 1)

    def pair_body(i, _):
        r = 2 * i
        _drain(emb_hbm, codes_v, rows_a, sem_a)
        _pool_round(codes_v, rows_a, out_v, out_hbm, emb_hbm, sem_a,
                    base + r * R)
        _fire(emb_hbm, codes_v, rows_a, sem_a, r + 2)
        _drain(emb_hbm, codes_v, rows_b, sem_b)
        _pool_round(codes_v, rows_b, out_v, out_hbm, emb_hbm, sem_b,
                    base + (r + 1) * R)
        _fire(emb_hbm, codes_v, rows_b, sem_b, r + 3)
        return 0

    lax.fori_loop(0, nrounds // 2 - 1, pair_body, 0)

    r = nrounds - 2
    _drain(emb_hbm, codes_v, rows_a, sem_a)
    _pool_round(codes_v, rows_a, out_v, out_hbm, emb_hbm, sem_a,
                base + r * R)
    _drain(emb_hbm, codes_v, rows_b, sem_b)
    _pool_round(codes_v, rows_b, out_v, out_hbm, emb_hbm, sem_b,
                base + (r + 1) * R)


def kernel(item_seq, pq_codes, emb_table):
    batch, hist = item_seq.shape
    n_queries = batch * hist
    info = plsc.get_sparse_core_info()
    num_workers = info.num_cores * info.num_subcores
    qpw = n_queries // num_workers
    assert qpw % STAGE == 0 and qpw % (2 * R) == 0

    mesh = plsc.VectorSubcoreMesh(core_axis_name="c", subcore_axis_name="s")
    run = pl.kernel(
        functools.partial(_sc_body, num_workers, n_queries),
        out_type=jax.ShapeDtypeStruct((n_queries, OUT_DIM), jnp.float32),
        mesh=mesh,
        scratch_types=[
            pltpu.VMEM((qpw,), jnp.int32),
            pltpu.VMEM((qpw, CODE_DIM), jnp.int32),
            pltpu.VMEM((R, CODE_DIM, OUT_DIM), jnp.float32),
            pltpu.VMEM((R, CODE_DIM, OUT_DIM), jnp.float32),
            pltpu.VMEM((R, OUT_DIM), jnp.float32),
            pltpu.SemaphoreType.DMA,
            pltpu.SemaphoreType.DMA,
            pltpu.SemaphoreType.DMA,
        ],
        compiler_params=pltpu.CompilerParams(use_tc_tiling_on_sc=False),
    )
    out = run(item_seq.reshape(n_queries), pq_codes, emb_table)
    return out.reshape(batch, hist, OUT_DIM)
